# baseline (speedup 1.0000x reference)
import jax
import jax.numpy as jnp
from jax.experimental import pallas as pl
from jax.experimental.pallas import tpu as pltpu

_CHUNK = 512


def _round_up(v, m):
    return ((v + m - 1) // m) * m


def _mlp_kernel(x_hbm, w1_hbm, b1_hbm, w2_hbm, b2_hbm, o_hbm,
                w1s, b1s, w2s, b2s, xb, ob, vmem_pad, wsem, xsem, osem, *,
                n_chunks):
    cp = pltpu.make_async_copy
    cp(w1_hbm, w1s, wsem.at[0]).start()
    cp(b1_hbm, b1s, wsem.at[1]).start()
    cp(w2_hbm, w2s, wsem.at[2]).start()
    cp(b2_hbm, b2s, wsem.at[3]).start()
    cp(x_hbm.at[pl.ds(0, _CHUNK), :], xb.at[0], xsem.at[0]).start()
    cp(w1_hbm, w1s, wsem.at[0]).wait()
    cp(b1_hbm, b1s, wsem.at[1]).wait()

    for c in range(n_chunks):
        s = c % 2
        if c + 1 < n_chunks:
            cp(x_hbm.at[pl.ds((c + 1) * _CHUNK, _CHUNK), :], xb.at[1 - s],
               xsem.at[1 - s]).start()
        cp(xb.at[s], xb.at[s], xsem.at[s]).wait()
        h = jnp.dot(xb[s], w1s[...], preferred_element_type=jnp.float32)
        h = h + b1s[...]
        h = h * jax.nn.sigmoid(h)
        if c == 0:
            cp(w2_hbm, w2s, wsem.at[2]).wait()
            cp(b2_hbm, b2s, wsem.at[3]).wait()
        res = jnp.dot(h, w2s[...], preferred_element_type=jnp.float32)
        res = res + b2s[...]
        if c >= 2:
            cp(ob.at[s], ob.at[s], osem.at[s]).wait()
        ob[s, :, :] = res.astype(ob.dtype)
        cp(ob.at[s], o_hbm.at[pl.ds(c * _CHUNK, _CHUNK), :],
           osem.at[s]).start()

    for s in range(min(2, n_chunks)):
        cp(ob.at[s], ob.at[s], osem.at[s]).wait()


def kernel(w1, b1, w2, b2, x):
    B, Din = x.shape
    D = w1.shape[1]

    Dp = _round_up(D, 128)
    Bp = _round_up(B, _CHUNK)

    xp = x if Bp == B else jnp.pad(x, ((0, Bp - B), (0, 0)))
    w1p = w1 if Dp == D else jnp.pad(w1, ((0, 0), (0, Dp - D)))
    w2p = w2 if Dp == D else jnp.pad(w2, ((0, Dp - D), (0, Dp - D)))
    b1p = (b1 if Dp == D else jnp.pad(b1, (0, Dp - D))).reshape(1, Dp)
    b2p = (b2 if Dp == D else jnp.pad(b2, (0, Dp - D))).reshape(1, Dp)

    import functools
    out = pl.pallas_call(
        functools.partial(_mlp_kernel, n_chunks=Bp // _CHUNK),
        out_shape=jax.ShapeDtypeStruct((Bp, Dp), x.dtype),
        in_specs=[
            pl.BlockSpec(memory_space=pl.ANY),
            pl.BlockSpec(memory_space=pl.ANY),
            pl.BlockSpec(memory_space=pl.ANY),
            pl.BlockSpec(memory_space=pl.ANY),
            pl.BlockSpec(memory_space=pl.ANY),
        ],
        out_specs=pl.BlockSpec(memory_space=pl.ANY),
        scratch_shapes=[
            pltpu.VMEM((Din, Dp), jnp.float32),
            pltpu.VMEM((1, Dp), jnp.float32),
            pltpu.VMEM((Dp, Dp), jnp.float32),
            pltpu.VMEM((1, Dp), jnp.float32),
            pltpu.VMEM((2, _CHUNK, Din), jnp.float32),
            pltpu.VMEM((2, _CHUNK, Dp), jnp.float32),
            pltpu.VMEM((9984, 1024), jnp.float32),
            pltpu.SemaphoreType.DMA((4,)),
            pltpu.SemaphoreType.DMA((2,)),
            pltpu.SemaphoreType.DMA((2,)),
        ],
        compiler_params=pltpu.CompilerParams(
            dimension_semantics=(),
            vmem_limit_bytes=60 * 1024 * 1024,
        ),
    )(xp, w1p, b1p, w2p, b2p)

    return out[:B, :D]


# speedup vs baseline: 1.1013x; 1.1013x over previous
import jax
import jax.numpy as jnp
from jax.experimental import pallas as pl
from jax.experimental.pallas import tpu as pltpu


def _round_up(v, m):
    return ((v + m - 1) // m) * m


def _mlp_kernel(x_ref, w1_ref, b1_ref, w2_hbm, b2_hbm, o_ref,
                w2s, b2s, vmem_pad, sems):
    first = pl.program_id(0) == 0

    @pl.when(first)
    def _start_w2():
        pltpu.make_async_copy(w2_hbm, w2s, sems.at[0]).start()
        pltpu.make_async_copy(b2_hbm, b2s, sems.at[1]).start()

    tm = x_ref.shape[0]
    sub = 512 if tm % 512 == 0 else tm
    for j in range(tm // sub):
        rows = pl.ds(j * sub, sub)
        h = jnp.dot(x_ref[rows, :], w1_ref[...],
                    preferred_element_type=jnp.float32)
        h = h + b1_ref[...]
        h = h * jax.nn.sigmoid(h)
        if j == 0:
            @pl.when(first)
            def _wait_w2():
                pltpu.make_async_copy(w2_hbm, w2s, sems.at[0]).wait()
                pltpu.make_async_copy(b2_hbm, b2s, sems.at[1]).wait()
        out = jnp.dot(h, w2s[...], preferred_element_type=jnp.float32)
        o_ref[rows, :] = (out + b2s[...]).astype(o_ref.dtype)


def kernel(w1, b1, w2, b2, x):
    B, Din = x.shape
    D = w1.shape[1]

    Dp = _round_up(D, 128)
    TM = 2048 if B % 4096 == 0 else _round_up(min(512, B), 8)
    Bp = _round_up(B, TM)

    xp = x if Bp == B else jnp.pad(x, ((0, Bp - B), (0, 0)))
    w1p = w1 if Dp == D else jnp.pad(w1, ((0, 0), (0, Dp - D)))
    w2p = w2 if Dp == D else jnp.pad(w2, ((0, Dp - D), (0, Dp - D)))
    b1p = (b1 if Dp == D else jnp.pad(b1, (0, Dp - D))).reshape(1, Dp)
    b2p = (b2 if Dp == D else jnp.pad(b2, (0, Dp - D))).reshape(1, Dp)

    out = pl.pallas_call(
        _mlp_kernel,
        out_shape=jax.ShapeDtypeStruct((Bp, Dp), x.dtype),
        grid=(Bp // TM,),
        in_specs=[
            pl.BlockSpec((TM, Din), lambda i: (i, 0)),
            pl.BlockSpec((Din, Dp), lambda i: (0, 0)),
            pl.BlockSpec((1, Dp), lambda i: (0, 0)),
            pl.BlockSpec(memory_space=pl.ANY),
            pl.BlockSpec(memory_space=pl.ANY),
        ],
        out_specs=pl.BlockSpec((TM, Dp), lambda i: (i, 0)),
        scratch_shapes=[
            pltpu.VMEM((Dp, Dp), jnp.float32),
            pltpu.VMEM((1, Dp), jnp.float32),
            pltpu.VMEM((3840, 1024), jnp.float32),
            pltpu.SemaphoreType.DMA((2,)),
        ],
        compiler_params=pltpu.CompilerParams(
            dimension_semantics=("arbitrary",),
            vmem_limit_bytes=60 * 1024 * 1024,
        ),
    )(xp, w1p, b1p, w2p, b2p)

    return out[:B, :D]


# speedup vs baseline: 1.1589x; 1.0523x over previous
import jax
import jax.numpy as jnp
from jax.experimental import pallas as pl
from jax.experimental.pallas import tpu as pltpu


def _round_up(v, m):
    return ((v + m - 1) // m) * m


def _mlp_kernel(x_ref, w1_ref, b1_ref, w2_ref, b2_ref, o_ref, vmem_pad):
    tm = x_ref.shape[0]
    sub = 512 if tm % 512 == 0 else tm
    for j in range(tm // sub):
        rows = pl.ds(j * sub, sub)
        h = jnp.dot(x_ref[rows, :], w1_ref[...],
                    preferred_element_type=jnp.float32)
        h = h + b1_ref[...]
        h = h * jax.nn.sigmoid(h)
        out = jnp.dot(h, w2_ref[...], preferred_element_type=jnp.float32)
        o_ref[rows, :] = (out + b2_ref[...]).astype(o_ref.dtype)


def kernel(w1, b1, w2, b2, x):
    B, Din = x.shape
    D = w1.shape[1]

    Dp = _round_up(D, 128)
    TM = 2048 if B % 4096 == 0 else _round_up(min(512, B), 8)
    Bp = _round_up(B, TM)

    xp = x if Bp == B else jnp.pad(x, ((0, Bp - B), (0, 0)))
    w1p = w1 if Dp == D else jnp.pad(w1, ((0, 0), (0, Dp - D)))
    w2p = w2 if Dp == D else jnp.pad(w2, ((0, Dp - D), (0, Dp - D)))
    b1p = (b1 if Dp == D else jnp.pad(b1, (0, Dp - D))).reshape(1, Dp)
    b2p = (b2 if Dp == D else jnp.pad(b2, (0, Dp - D))).reshape(1, Dp)

    out = pl.pallas_call(
        _mlp_kernel,
        out_shape=jax.ShapeDtypeStruct((Bp, Dp), x.dtype),
        grid=(Bp // TM,),
        in_specs=[
            pl.BlockSpec((TM, Din), lambda i: (i, 0)),
            pl.BlockSpec((Din, Dp), lambda i: (0, 0)),
            pl.BlockSpec((1, Dp), lambda i: (0, 0)),
            pl.BlockSpec((Dp, Dp), lambda i: (0, 0)),
            pl.BlockSpec((1, Dp), lambda i: (0, 0)),
        ],
        out_specs=pl.BlockSpec((TM, Dp), lambda i: (i, 0)),
        scratch_shapes=[
            pltpu.VMEM((3840, 1024), jnp.float32),
        ],
        compiler_params=pltpu.CompilerParams(
            dimension_semantics=("parallel",),
            vmem_limit_bytes=60 * 1024 * 1024,
        ),
    )(xp, w1p, b1p, w2p, b2p)

    return out[:B, :D]


# speedup vs baseline: 1.1613x; 1.0020x over previous
import jax
import jax.numpy as jnp
from jax.experimental import pallas as pl
from jax.experimental.pallas import tpu as pltpu


def _round_up(v, m):
    return ((v + m - 1) // m) * m


def _mlp_kernel(x_ref, w1_ref, b1_ref, w2_ref, b2_ref, o_ref, vmem_pad):
    tm = x_ref.shape[0]
    sub = 1024 if tm % 1024 == 0 else tm
    for j in range(tm // sub):
        rows = pl.ds(j * sub, sub)
        h = jnp.dot(x_ref[rows, :], w1_ref[...],
                    preferred_element_type=jnp.float32)
        h = h + b1_ref[...]
        h = h * jax.nn.sigmoid(h)
        out = jnp.dot(h, w2_ref[...], preferred_element_type=jnp.float32)
        o_ref[rows, :] = (out + b2_ref[...]).astype(o_ref.dtype)


def kernel(w1, b1, w2, b2, x):
    B, Din = x.shape
    D = w1.shape[1]

    Dp = _round_up(D, 128)
    TM = 2048 if B % 4096 == 0 else _round_up(min(512, B), 8)
    Bp = _round_up(B, TM)

    xp = x if Bp == B else jnp.pad(x, ((0, Bp - B), (0, 0)))
    w1p = w1 if Dp == D else jnp.pad(w1, ((0, 0), (0, Dp - D)))
    w2p = w2 if Dp == D else jnp.pad(w2, ((0, Dp - D), (0, Dp - D)))
    b1p = (b1 if Dp == D else jnp.pad(b1, (0, Dp - D))).reshape(1, Dp)
    b2p = (b2 if Dp == D else jnp.pad(b2, (0, Dp - D))).reshape(1, Dp)

    out = pl.pallas_call(
        _mlp_kernel,
        out_shape=jax.ShapeDtypeStruct((Bp, Dp), x.dtype),
        grid=(Bp // TM,),
        in_specs=[
            pl.BlockSpec((TM, Din), lambda i: (i, 0)),
            pl.BlockSpec((Din, Dp), lambda i: (0, 0)),
            pl.BlockSpec((1, Dp), lambda i: (0, 0)),
            pl.BlockSpec((Dp, Dp), lambda i: (0, 0)),
            pl.BlockSpec((1, Dp), lambda i: (0, 0)),
        ],
        out_specs=pl.BlockSpec((TM, Dp), lambda i: (i, 0)),
        scratch_shapes=[
            pltpu.VMEM((3840, 1024), jnp.float32),
        ],
        compiler_params=pltpu.CompilerParams(
            dimension_semantics=("parallel",),
            vmem_limit_bytes=60 * 1024 * 1024,
        ),
    )(xp, w1p, b1p, w2p, b2p)

    return out[:B, :D]
